# Initial kernel scaffold; baseline (speedup 1.0000x reference)
import jax
import jax.numpy as jnp
from jax.experimental import pallas as pl


def kernel(inputs, router_w, router_b, thr_w, thr_b, A_ws, B_ws):
    raise NotImplementedError("write your pallas kernel here")


# speedup vs baseline: 9.5711x; 9.5711x over previous
import functools

import jax
import jax.numpy as jnp
from jax.experimental import pallas as pl
from jax.experimental.pallas import tpu as pltpu

E = 8
R = 32
D = 2048
O = 2048
T = 8192
ER = E * R
NCOLS = ER + 128


def _fused_kernel(x_ref, wcat_ref, bcat_ref, rb_ref, tb_ref, out_ref):
    x = x_ref[...]
    h_full = jnp.dot(x, wcat_ref[...], preferred_element_type=jnp.float32)
    logits = h_full[:, ER:ER + E] + rb_ref[...]
    thr_logit = h_full[:, ER + E:ER + E + 1] + tb_ref[...]
    gate = jax.nn.softmax(logits, axis=-1)
    thr = jax.nn.sigmoid(thr_logit) * (1.0 / E)
    adapted = gate - thr
    w = jnp.where(adapted >= 0.0, adapted, 0.0)
    s = jnp.sum(w, axis=-1, keepdims=True)
    s = jnp.where(s == 0.0, 1.0, s)
    w = w / s
    gid = jax.lax.broadcasted_iota(jnp.int32, (1, ER), 1) // R
    w_exp = jnp.zeros((x.shape[0], ER), dtype=jnp.float32)
    for e in range(E):
        w_exp = w_exp + jnp.where(gid == e, 1.0, 0.0) * w[:, e:e + 1]
    hw = h_full[:, :ER] * w_exp
    out_ref[...] = jnp.dot(hw, bcat_ref[...], preferred_element_type=jnp.float32)


@functools.partial(jax.jit, static_argnames=("block_t",))
def _run(inputs, router_w, router_b, thr_w, thr_b, A_ws, B_ws, block_t=512):
    a_cat = jnp.transpose(A_ws, (2, 0, 1)).reshape(D, ER)
    pad = jnp.zeros((D, NCOLS - ER - E - 1), dtype=jnp.float32)
    w_cat = jnp.concatenate([a_cat, router_w.T, thr_w.T, pad], axis=1)
    b_cat = jnp.transpose(B_ws, (0, 2, 1)).reshape(ER, O)
    rb = router_b.reshape(1, E)
    tb = thr_b.reshape(1, 1)

    grid = (T // block_t,)
    return pl.pallas_call(
        _fused_kernel,
        grid=grid,
        in_specs=[
            pl.BlockSpec((block_t, D), lambda i: (i, 0)),
            pl.BlockSpec((D, NCOLS), lambda i: (0, 0)),
            pl.BlockSpec((ER, O), lambda i: (0, 0)),
            pl.BlockSpec((1, E), lambda i: (0, 0)),
            pl.BlockSpec((1, 1), lambda i: (0, 0)),
        ],
        out_specs=pl.BlockSpec((block_t, O), lambda i: (i, 0)),
        out_shape=jax.ShapeDtypeStruct((T, O), jnp.float32),
        compiler_params=pltpu.CompilerParams(
            dimension_semantics=("parallel",),
        ),
    )(inputs, w_cat, b_cat, rb, tb)


def kernel(inputs, router_w, router_b, thr_w, thr_b, A_ws, B_ws):
    return _run(inputs, router_w, router_b, thr_w, thr_b, A_ws, B_ws)
